# baseline (speedup 1.0000x reference)
import functools
import math

import jax
import jax.numpy as jnp
from jax import lax
from jax.experimental import pallas as pl
from jax.experimental.pallas import tpu as pltpu
from jax.experimental.pallas import tpu_sc as plsc

MAX_TOKENS = 1024
DIM = 1024
SCALE = 1.0 / math.sqrt(DIM)

NUM_CORES = 2
NUM_SUBCORES = 16
NUM_WORKERS = NUM_CORES * NUM_SUBCORES
LANES = 16

ROWS_PER_WORKER = MAX_TOKENS // NUM_WORKERS
VECS_PER_ROW = DIM // LANES

_mesh = plsc.VectorSubcoreMesh(core_axis_name="c", subcore_axis_name="s")


@functools.partial(
    pl.kernel,
    mesh=_mesh,
    out_type=jax.ShapeDtypeStruct((MAX_TOKENS, DIM), jnp.float32),
    scratch_types=[
        pltpu.VMEM((ROWS_PER_WORKER, DIM), jnp.float32),
        pltpu.VMEM((ROWS_PER_WORKER, DIM), jnp.float32),
    ],
)
def _scaled_copy(embed_hbm, out_hbm, buf_in, buf_out):
    wid = lax.axis_index("s") * NUM_CORES + lax.axis_index("c")
    base = wid * ROWS_PER_WORKER
    pltpu.sync_copy(embed_hbm.at[pl.ds(base, ROWS_PER_WORKER)], buf_in)

    @plsc.parallel_loop(0, ROWS_PER_WORKER * VECS_PER_ROW, unroll=4)
    def _(i):
        r = lax.shift_right_logical(i, 6)
        c = lax.shift_left(lax.bitwise_and(i, VECS_PER_ROW - 1), 4)
        sl = pl.ds(pl.multiple_of(c, LANES), LANES)
        buf_out[r, sl] = buf_in[r, sl] * SCALE

    pltpu.sync_copy(buf_out, out_hbm.at[pl.ds(base, ROWS_PER_WORKER)])


def kernel(x, embed):
    del x
    return _scaled_copy(embed)


# speedup vs baseline: 1.0102x; 1.0102x over previous
import functools
import math

import jax
import jax.numpy as jnp
from jax import lax
from jax.experimental import pallas as pl
from jax.experimental.pallas import tpu as pltpu
from jax.experimental.pallas import tpu_sc as plsc

MAX_TOKENS = 1024
DIM = 1024
SCALE = 1.0 / math.sqrt(DIM)

NUM_CORES = 2
NUM_SUBCORES = 16
NUM_WORKERS = NUM_CORES * NUM_SUBCORES
LANES = 16

ROWS_PER_WORKER = MAX_TOKENS // NUM_WORKERS
VECS_PER_ROW = DIM // LANES
NCHUNK = 4
CHUNK_ROWS = ROWS_PER_WORKER // NCHUNK
CHUNK_VECS = CHUNK_ROWS * VECS_PER_ROW

_mesh = plsc.VectorSubcoreMesh(core_axis_name="c", subcore_axis_name="s")


@functools.partial(
    pl.kernel,
    mesh=_mesh,
    out_type=jax.ShapeDtypeStruct((MAX_TOKENS, DIM), jnp.float32),
    scratch_types=[
        pltpu.VMEM((2, CHUNK_ROWS, DIM), jnp.float32),
        pltpu.VMEM((2, CHUNK_ROWS, DIM), jnp.float32),
        pltpu.SemaphoreType.DMA,
        pltpu.SemaphoreType.DMA,
        pltpu.SemaphoreType.DMA,
        pltpu.SemaphoreType.DMA,
    ],
)
def _scaled_copy(embed_hbm, out_hbm, buf_in, buf_out, si0, si1, so0, so1):
    wid = lax.axis_index("s") * NUM_CORES + lax.axis_index("c")
    base = wid * ROWS_PER_WORKER
    sin = (si0, si1)
    sout = (so0, so1)

    def start_in(k):
        return pltpu.async_copy(
            embed_hbm.at[pl.ds(base + k * CHUNK_ROWS, CHUNK_ROWS)],
            buf_in.at[k % 2],
            sin[k % 2],
        )

    def start_out(k):
        return pltpu.async_copy(
            buf_out.at[k % 2],
            out_hbm.at[pl.ds(base + k * CHUNK_ROWS, CHUNK_ROWS)],
            sout[k % 2],
        )

    in_handles = [start_in(0), start_in(1)]
    out_handles = [None, None]
    for k in range(NCHUNK):
        b = k % 2
        in_handles[b].wait()
        if out_handles[b] is not None:
            out_handles[b].wait()

        src = buf_in.at[b]
        dst = buf_out.at[b]

        @plsc.parallel_loop(0, CHUNK_VECS, unroll=4)
        def _(i):
            r = lax.shift_right_logical(i, 6)
            c = lax.shift_left(lax.bitwise_and(i, VECS_PER_ROW - 1), 4)
            sl = pl.ds(pl.multiple_of(c, LANES), LANES)
            dst[r, sl] = src[r, sl] * SCALE

        out_handles[b] = start_out(k)
        if k + 2 < NCHUNK:
            in_handles[b] = start_in(k + 2)
    out_handles[0].wait()
    out_handles[1].wait()


def kernel(x, embed):
    del x
    return _scaled_copy(embed)
